# baseline (speedup 1.0000x reference)
import jax
import jax.numpy as jnp
from jax import lax
from jax.experimental import pallas as pl
from jax.experimental.pallas import tpu as pltpu
from jax.experimental.pallas import tpu_sc as plsc

F32 = jnp.float32

NUM_CORES = 2
NUM_SUBCORES = 16
NW = NUM_CORES * NUM_SUBCORES
K = 128
DEGW = 8
NBUF = 4

_SC_PARAMS = pltpu.CompilerParams(use_tc_tiling_on_sc=False)


def _gather_scatter_loop(tab, sidx, didx, bufs, gsems, ssems, acc, nch):
    pltpu.async_copy(tab.at[sidx.at[0]], bufs[0], gsems[0])
    pltpu.async_copy(tab.at[sidx.at[1]], bufs[1], gsems[1])

    def step(j, carry):
        @pl.when(j + 2 < nch)
        def _():
            for a in range(NBUF):
                def refill(a=a):
                    @pl.when(j >= 2)
                    def _():
                        pltpu.make_async_copy(
                            bufs[a], acc.at[didx.at[j - 2]], ssems[a]).wait()
                    pltpu.async_copy(tab.at[sidx.at[j + 2]], bufs[a],
                                     gsems[a])
                pl.when((j + 2) % NBUF == a)(refill)

        for b in range(NBUF):
            def consume(b=b):
                pltpu.make_async_copy(tab.at[sidx.at[j]], bufs[b],
                                      gsems[b]).wait()
                pltpu.async_copy(bufs[b], acc.at[didx.at[j]], ssems[b],
                                 add=True)
            pl.when(j % NBUF == b)(consume)
        return carry

    lax.fori_loop(0, nch, step, 0)
    for jj in range(nch - NBUF, nch):
        pltpu.make_async_copy(bufs[jj % NBUF], acc.at[didx.at[jj]],
                              ssems[jj % NBUF]).wait()


def _make_sc_layer1(n_pad, rows_per_tile, nch2, hw):
    mesh = plsc.VectorSubcoreMesh(
        core_axis_name="c", subcore_axis_name="s",
        num_cores=NUM_CORES, num_subcores=NUM_SUBCORES)

    def body(tab0, tab1, srcr, dstr, zeros_w, out,
             sidx, didx, b0, b1, b2, b3, acc,
             g0, g1, g2, g3, s0, s1, s2, s3):
        bufs = (b0, b1, b2, b3)
        gsems = (g0, g1, g2, g3)
        ssems = (s0, s1, s2, s3)
        c = lax.axis_index("c")
        s = lax.axis_index("s")
        row0 = s * rows_per_tile
        sl = pl.ds(row0, rows_per_tile)

        pltpu.sync_copy(zeros_w, acc.at[sl])
        pltpu.sync_copy(srcr.at[s], sidx)
        pltpu.sync_copy(dstr.at[s], didx)
        plsc.subcore_barrier()

        @pl.when(c == 0)
        def _():
            _gather_scatter_loop(tab0, sidx, didx, bufs, gsems, ssems, acc,
                                 nch2)

        @pl.when(c == 1)
        def _():
            _gather_scatter_loop(tab1, sidx, didx, bufs, gsems, ssems, acc,
                                 nch2)

        plsc.subcore_barrier()
        pltpu.sync_copy(acc.at[sl], out.at[c, sl])

    return pl.kernel(
        body,
        out_type=jax.ShapeDtypeStruct((NUM_CORES, n_pad, hw), F32),
        mesh=mesh,
        compiler_params=_SC_PARAMS,
        scratch_types=(
            [pltpu.VMEM((nch2, K), jnp.int32)] * 2 +
            [pltpu.VMEM((K, hw), F32)] * NBUF +
            [pltpu.VMEM_SHARED((n_pad, hw), F32)] +
            [pltpu.SemaphoreType.DMA] * (2 * NBUF)))


def _make_sc_layer2(n_pad, rows_per_tile, nch, w):
    mesh = plsc.VectorSubcoreMesh(
        core_axis_name="c", subcore_axis_name="s",
        num_cores=NUM_CORES, num_subcores=NUM_SUBCORES)

    def body(tab, srcr, dstr, zeros_w, out, sidx, didx, b0, b1, b2, b3, acc,
             g0, g1, g2, g3, s0, s1, s2, s3):
        bufs = (b0, b1, b2, b3)
        gsems = (g0, g1, g2, g3)
        ssems = (s0, s1, s2, s3)
        c = lax.axis_index("c")
        s = lax.axis_index("s")
        wid = c * NUM_SUBCORES + s
        row0 = s * rows_per_tile
        sl = pl.ds(row0, rows_per_tile)

        pltpu.sync_copy(zeros_w, acc.at[sl])
        pltpu.sync_copy(srcr.at[wid], sidx)
        pltpu.sync_copy(dstr.at[wid], didx)
        plsc.subcore_barrier()

        _gather_scatter_loop(tab, sidx, didx, bufs, gsems, ssems, acc, nch)

        plsc.subcore_barrier()
        pltpu.sync_copy(acc.at[sl], out.at[c, sl])

    return pl.kernel(
        body,
        out_type=jax.ShapeDtypeStruct((NUM_CORES, n_pad, w), F32),
        mesh=mesh,
        compiler_params=_SC_PARAMS,
        scratch_types=(
            [pltpu.VMEM((nch, K), jnp.int32)] * 2 +
            [pltpu.VMEM((K, w), F32)] * NBUF +
            [pltpu.VMEM_SHARED((n_pad, w), F32)] +
            [pltpu.SemaphoreType.DMA] * (2 * NBUF)))


def _tc_lin(x, wl, wr, bl, bn):
    n, d = x.shape
    h = wl.shape[0]
    hh = h // 2
    dn = (((1,), (1,)), ((), ()))

    def body(x_ref, wl_ref, wr_ref, bl_ref, ya_ref, yb_ref, z_ref):
        xb = x_ref[...]
        y = lax.dot_general(xb, wl_ref[...], dn, preferred_element_type=F32)
        ones = jnp.ones((y.shape[0], DEGW), F32)
        ya_ref[...] = jnp.concatenate([y[:, :hh], ones], axis=1)
        yb_ref[...] = jnp.concatenate([y[:, hh:], ones], axis=1)
        z_ref[...] = lax.dot_general(xb, wr_ref[...], dn,
                                     preferred_element_type=F32) + bl_ref[...]

    grid = (n // bn,)
    hw = hh + DEGW
    return pl.pallas_call(
        body,
        grid=grid,
        in_specs=[
            pl.BlockSpec((bn, d), lambda i: (i, 0)),
            pl.BlockSpec((h, d), lambda i: (0, 0)),
            pl.BlockSpec((h, d), lambda i: (0, 0)),
            pl.BlockSpec((1, h), lambda i: (0, 0)),
        ],
        out_specs=[
            pl.BlockSpec((bn, hw), lambda i: (i, 0)),
            pl.BlockSpec((bn, hw), lambda i: (i, 0)),
            pl.BlockSpec((bn, h), lambda i: (i, 0)),
        ],
        out_shape=[
            jax.ShapeDtypeStruct((n, hw), F32),
            jax.ShapeDtypeStruct((n, hw), F32),
            jax.ShapeDtypeStruct((n, h), F32),
        ],
    )(x, wl, wr, bl)


def _tc_combine_stats(agg, z1, bn):
    n, h = z1.shape
    hh = h // 2
    hw = agg.shape[2]

    def body(a_ref, z_ref, h_ref, st_ref, dg_ref):
        i = pl.program_id(0)
        a0 = a_ref[0]
        a1 = a_ref[1]
        a = jnp.concatenate([a0[:, :hh], a1[:, :hh]], axis=1)
        dg = a0[:, hh:hh + 1]
        hp = a / jnp.maximum(dg, 1.0) + z_ref[...]
        h_ref[...] = hp
        dg_ref[...] = dg
        s1 = jnp.sum(hp, axis=0, keepdims=True)
        s2 = jnp.sum(hp * hp, axis=0, keepdims=True)

        @pl.when(i == 0)
        def _():
            st_ref[...] = jnp.zeros_like(st_ref)

        st_ref[...] += jnp.concatenate([s1, s2], axis=0)

    grid = (n // bn,)
    return pl.pallas_call(
        body,
        grid=grid,
        in_specs=[
            pl.BlockSpec((2, bn, hw), lambda i: (0, i, 0)),
            pl.BlockSpec((bn, h), lambda i: (i, 0)),
        ],
        out_specs=[
            pl.BlockSpec((bn, h), lambda i: (i, 0)),
            pl.BlockSpec((2, h), lambda i: (0, 0)),
            pl.BlockSpec((bn, 1), lambda i: (i, 0)),
        ],
        out_shape=[
            jax.ShapeDtypeStruct((n, h), F32),
            jax.ShapeDtypeStruct((2, h), F32),
            jax.ShapeDtypeStruct((n, 1), F32),
        ],
    )(agg, z1)


def _tc_bn_lin(hpre, stats, gamma, beta, wl2, wr2, bl2, bn):
    n, h = hpre.shape
    c = wl2.shape[0]
    dn = (((1,), (1,)), ((), ()))
    inv_n = 1.0 / n

    def body(h_ref, st_ref, g_ref, b_ref, wl_ref, wr_ref, bl_ref,
             y_ref, z_ref):
        mu = st_ref[0:1] * inv_n
        var = st_ref[1:2] * inv_n - mu * mu
        scale = g_ref[...] * lax.rsqrt(var + 1e-5)
        shift = b_ref[...] - mu * scale
        hn = jnp.maximum(h_ref[...] * scale + shift, 0.0)
        y_ref[...] = lax.dot_general(hn, wl_ref[...], dn,
                                     preferred_element_type=F32)
        z_ref[...] = lax.dot_general(hn, wr_ref[...], dn,
                                     preferred_element_type=F32) + bl_ref[...]

    grid = (n // bn,)
    return pl.pallas_call(
        body,
        grid=grid,
        in_specs=[
            pl.BlockSpec((bn, h), lambda i: (i, 0)),
            pl.BlockSpec((2, h), lambda i: (0, 0)),
            pl.BlockSpec((1, h), lambda i: (0, 0)),
            pl.BlockSpec((1, h), lambda i: (0, 0)),
            pl.BlockSpec((c, h), lambda i: (0, 0)),
            pl.BlockSpec((c, h), lambda i: (0, 0)),
            pl.BlockSpec((1, c), lambda i: (0, 0)),
        ],
        out_specs=[
            pl.BlockSpec((bn, c), lambda i: (i, 0)),
            pl.BlockSpec((bn, c), lambda i: (i, 0)),
        ],
        out_shape=[
            jax.ShapeDtypeStruct((n, c), F32),
            jax.ShapeDtypeStruct((n, c), F32),
        ],
    )(hpre, stats, gamma, beta, wl2, wr2, bl2)


def _tc_final(agg, deg, z2, bn):
    n, c = z2.shape

    def body(a_ref, d_ref, z_ref, o_ref):
        a = a_ref[0] + a_ref[1]
        o_ref[...] = a / jnp.maximum(d_ref[...], 1.0) + z_ref[...]

    grid = (n // bn,)
    return pl.pallas_call(
        body,
        grid=grid,
        in_specs=[
            pl.BlockSpec((2, bn, c), lambda i: (0, i, 0)),
            pl.BlockSpec((bn, 1), lambda i: (i, 0)),
            pl.BlockSpec((bn, c), lambda i: (i, 0)),
        ],
        out_specs=pl.BlockSpec((bn, c), lambda i: (i, 0)),
        out_shape=jax.ShapeDtypeStruct((n, c), F32),
    )(agg, deg, z2)


def kernel(x, edge_index, Wl1, bl1, Wr1, gamma, beta, Wl2, bl2, Wr2):
    n, d = x.shape
    e = edge_index.shape[1]
    h = Wl1.shape[0]
    c = Wl2.shape[0]

    nch = NBUF * (-(-e // (NW * K * NBUF)))
    nch2 = 2 * nch
    e_pad = NW * K * nch
    rows_per_tile = -(-(n + 1) // (NUM_SUBCORES * 8)) * 8
    n_pad = NUM_SUBCORES * rows_per_tile

    src = edge_index[0]
    dst = edge_index[1]
    pad = e_pad - e
    if pad:
        src = jnp.concatenate([src, jnp.zeros((pad,), jnp.int32)])
        dst = jnp.concatenate([dst, jnp.full((pad,), n, jnp.int32)])
    srcr16 = src.reshape(NUM_SUBCORES, nch2, K)
    dstr16 = dst.reshape(NUM_SUBCORES, nch2, K)
    srcr32 = src.reshape(NW, nch, K)
    dstr32 = dst.reshape(NW, nch, K)

    hw = h // 2 + DEGW
    zeros_hw = jnp.zeros((rows_per_tile, hw), F32)
    zeros_c = jnp.zeros((rows_per_tile, c), F32)

    bn = 1000 if n % 1000 == 0 else 8 * (n // 8)

    y1a, y1b, z1 = _tc_lin(x, Wl1, Wr1, bl1.reshape(1, h), bn)
    seg1 = _make_sc_layer1(n_pad, rows_per_tile, nch2, hw)
    agg1 = seg1(y1a, y1b, srcr16, dstr16, zeros_hw)

    hpre, stats, deg = _tc_combine_stats(agg1, z1, bn)
    y2, z2 = _tc_bn_lin(hpre, stats, gamma.reshape(1, h), beta.reshape(1, h),
                        Wl2, Wr2, bl2.reshape(1, c), bn)

    seg2 = _make_sc_layer2(n_pad, rows_per_tile, nch, c)
    agg2 = seg2(y2, srcr32, dstr32, zeros_c)
    out = _tc_final(agg2, deg, z2, bn)
    return out


# speedup vs baseline: 1.6408x; 1.6408x over previous
import jax
import jax.numpy as jnp
from jax import lax
from jax.experimental import pallas as pl
from jax.experimental.pallas import tpu as pltpu
from jax.experimental.pallas import tpu_sc as plsc

F32 = jnp.float32

NUM_CORES = 2
NUM_SUBCORES = 16
NW = NUM_CORES * NUM_SUBCORES
K = 128
DEGW = 8
NBUF = 4

_SC_PARAMS = pltpu.CompilerParams(use_tc_tiling_on_sc=False)


def _gather_scatter_loop(tab, sidx, didx, bufs, gsems, ssems, acc, nch,
                         deg=None):
    if deg is not None:
        ones_v, dacc, dsem = deg
    pltpu.async_copy(tab.at[sidx.at[0]], bufs[0], gsems[0])
    pltpu.async_copy(tab.at[sidx.at[1]], bufs[1], gsems[1])

    def step(j, carry):
        @pl.when(j + 2 < nch)
        def _():
            for a in range(NBUF):
                def refill(a=a):
                    @pl.when(j >= 2)
                    def _():
                        pltpu.make_async_copy(
                            bufs[a], acc.at[didx.at[j - 2]], ssems[a]).wait()
                    pltpu.async_copy(tab.at[sidx.at[j + 2]], bufs[a],
                                     gsems[a])
                pl.when((j + 2) % NBUF == a)(refill)

        for b in range(NBUF):
            def consume(b=b):
                pltpu.make_async_copy(tab.at[sidx.at[j]], bufs[b],
                                      gsems[b]).wait()
                pltpu.async_copy(bufs[b], acc.at[didx.at[j]], ssems[b],
                                 add=True)
            pl.when(j % NBUF == b)(consume)

        if deg is not None:
            @pl.when(j >= 1)
            def _():
                pltpu.make_async_copy(ones_v, dacc.at[didx.at[j - 1]],
                                      dsem).wait()
            pltpu.async_copy(ones_v, dacc.at[didx.at[j]], dsem, add=True)
        return carry

    lax.fori_loop(0, nch, step, 0)
    for jj in range(nch - NBUF, nch):
        pltpu.make_async_copy(bufs[jj % NBUF], acc.at[didx.at[jj]],
                              ssems[jj % NBUF]).wait()
    if deg is not None:
        pltpu.make_async_copy(ones_v, dacc.at[didx.at[nch - 1]], dsem).wait()


def _make_sc_layer1(n_pad, rows_per_tile, nch2, hw):
    mesh = plsc.VectorSubcoreMesh(
        core_axis_name="c", subcore_axis_name="s",
        num_cores=NUM_CORES, num_subcores=NUM_SUBCORES)

    def body(tab0, tab1, srcr, dstr, zeros_w, zeros_d, ones_h, out, deg_out,
             sidx, didx, b0, b1, b2, b3, ones_v, acc, dacc,
             g0, g1, g2, g3, s0, s1, s2, s3, dsem):
        bufs = (b0, b1, b2, b3)
        gsems = (g0, g1, g2, g3)
        ssems = (s0, s1, s2, s3)
        c = lax.axis_index("c")
        s = lax.axis_index("s")
        row0 = s * rows_per_tile
        sl = pl.ds(row0, rows_per_tile)

        pltpu.sync_copy(zeros_w, acc.at[sl])
        pltpu.sync_copy(zeros_d, dacc.at[sl])
        pltpu.sync_copy(ones_h, ones_v)
        pltpu.sync_copy(srcr.at[s], sidx)
        pltpu.sync_copy(dstr.at[s], didx)
        plsc.subcore_barrier()

        deg = (ones_v, dacc, dsem)

        @pl.when(c == 0)
        def _():
            _gather_scatter_loop(tab0, sidx, didx, bufs, gsems, ssems, acc,
                                 nch2, deg)

        @pl.when(c == 1)
        def _():
            _gather_scatter_loop(tab1, sidx, didx, bufs, gsems, ssems, acc,
                                 nch2, deg)

        plsc.subcore_barrier()
        pltpu.sync_copy(acc.at[sl], out.at[c, sl])
        pltpu.sync_copy(dacc.at[sl], deg_out.at[c, sl])

    return pl.kernel(
        body,
        out_type=[
            jax.ShapeDtypeStruct((NUM_CORES, n_pad, hw), F32),
            jax.ShapeDtypeStruct((NUM_CORES, n_pad, DEGW), F32),
        ],
        mesh=mesh,
        compiler_params=_SC_PARAMS,
        scratch_types=(
            [pltpu.VMEM((nch2, K), jnp.int32)] * 2 +
            [pltpu.VMEM((K, hw), F32)] * NBUF +
            [pltpu.VMEM((K, DEGW), F32)] +
            [pltpu.VMEM_SHARED((n_pad, hw), F32),
             pltpu.VMEM_SHARED((n_pad, DEGW), F32)] +
            [pltpu.SemaphoreType.DMA] * (2 * NBUF + 1)))


def _make_sc_layer2(n_pad, rows_per_tile, nch, w):
    mesh = plsc.VectorSubcoreMesh(
        core_axis_name="c", subcore_axis_name="s",
        num_cores=NUM_CORES, num_subcores=NUM_SUBCORES)

    def body(tab, srcr, dstr, zeros_w, out, sidx, didx, b0, b1, b2, b3, acc,
             g0, g1, g2, g3, s0, s1, s2, s3):
        bufs = (b0, b1, b2, b3)
        gsems = (g0, g1, g2, g3)
        ssems = (s0, s1, s2, s3)
        c = lax.axis_index("c")
        s = lax.axis_index("s")
        wid = c * NUM_SUBCORES + s
        row0 = s * rows_per_tile
        sl = pl.ds(row0, rows_per_tile)

        pltpu.sync_copy(zeros_w, acc.at[sl])
        pltpu.sync_copy(srcr.at[wid], sidx)
        pltpu.sync_copy(dstr.at[wid], didx)
        plsc.subcore_barrier()

        _gather_scatter_loop(tab, sidx, didx, bufs, gsems, ssems, acc, nch)

        plsc.subcore_barrier()
        pltpu.sync_copy(acc.at[sl], out.at[c, sl])

    return pl.kernel(
        body,
        out_type=jax.ShapeDtypeStruct((NUM_CORES, n_pad, w), F32),
        mesh=mesh,
        compiler_params=_SC_PARAMS,
        scratch_types=(
            [pltpu.VMEM((nch, K), jnp.int32)] * 2 +
            [pltpu.VMEM((K, w), F32)] * NBUF +
            [pltpu.VMEM_SHARED((n_pad, w), F32)] +
            [pltpu.SemaphoreType.DMA] * (2 * NBUF)))


def _tc_lin(x, wl, wr, bl, bn):
    n, d = x.shape
    h = wl.shape[0]
    hh = h // 2
    dn = (((1,), (1,)), ((), ()))

    def body(x_ref, wl_ref, wr_ref, bl_ref, ya_ref, yb_ref, z_ref):
        xb = x_ref[...]
        y = lax.dot_general(xb, wl_ref[...], dn, preferred_element_type=F32)
        ya_ref[...] = y[:, :hh]
        yb_ref[...] = y[:, hh:]
        z_ref[...] = lax.dot_general(xb, wr_ref[...], dn,
                                     preferred_element_type=F32) + bl_ref[...]

    grid = (n // bn,)
    hw = hh
    return pl.pallas_call(
        body,
        grid=grid,
        in_specs=[
            pl.BlockSpec((bn, d), lambda i: (i, 0)),
            pl.BlockSpec((h, d), lambda i: (0, 0)),
            pl.BlockSpec((h, d), lambda i: (0, 0)),
            pl.BlockSpec((1, h), lambda i: (0, 0)),
        ],
        out_specs=[
            pl.BlockSpec((bn, hw), lambda i: (i, 0)),
            pl.BlockSpec((bn, hw), lambda i: (i, 0)),
            pl.BlockSpec((bn, h), lambda i: (i, 0)),
        ],
        out_shape=[
            jax.ShapeDtypeStruct((n, hw), F32),
            jax.ShapeDtypeStruct((n, hw), F32),
            jax.ShapeDtypeStruct((n, h), F32),
        ],
    )(x, wl, wr, bl)


def _tc_combine_stats(agg, degs, z1, bn):
    n, h = z1.shape
    hw = agg.shape[2]

    def body(a_ref, d_ref, z_ref, h_ref, st_ref, dg_ref):
        i = pl.program_id(0)
        a = jnp.concatenate([a_ref[0], a_ref[1]], axis=1)
        dg = d_ref[0][:, 0:1]
        hp = a / jnp.maximum(dg, 1.0) + z_ref[...]
        h_ref[...] = hp
        dg_ref[...] = dg
        s1 = jnp.sum(hp, axis=0, keepdims=True)
        s2 = jnp.sum(hp * hp, axis=0, keepdims=True)

        @pl.when(i == 0)
        def _():
            st_ref[...] = jnp.zeros_like(st_ref)

        st_ref[...] += jnp.concatenate([s1, s2], axis=0)

    grid = (n // bn,)
    return pl.pallas_call(
        body,
        grid=grid,
        in_specs=[
            pl.BlockSpec((2, bn, hw), lambda i: (0, i, 0)),
            pl.BlockSpec((1, bn, DEGW), lambda i: (0, i, 0)),
            pl.BlockSpec((bn, h), lambda i: (i, 0)),
        ],
        out_specs=[
            pl.BlockSpec((bn, h), lambda i: (i, 0)),
            pl.BlockSpec((2, h), lambda i: (0, 0)),
            pl.BlockSpec((bn, 1), lambda i: (i, 0)),
        ],
        out_shape=[
            jax.ShapeDtypeStruct((n, h), F32),
            jax.ShapeDtypeStruct((2, h), F32),
            jax.ShapeDtypeStruct((n, 1), F32),
        ],
    )(agg, degs, z1)


def _tc_bn_lin(hpre, stats, gamma, beta, wl2, wr2, bl2, bn):
    n, h = hpre.shape
    c = wl2.shape[0]
    dn = (((1,), (1,)), ((), ()))
    inv_n = 1.0 / n

    def body(h_ref, st_ref, g_ref, b_ref, wl_ref, wr_ref, bl_ref,
             y_ref, z_ref):
        mu = st_ref[0:1] * inv_n
        var = st_ref[1:2] * inv_n - mu * mu
        scale = g_ref[...] * lax.rsqrt(var + 1e-5)
        shift = b_ref[...] - mu * scale
        hn = jnp.maximum(h_ref[...] * scale + shift, 0.0)
        y_ref[...] = lax.dot_general(hn, wl_ref[...], dn,
                                     preferred_element_type=F32)
        z_ref[...] = lax.dot_general(hn, wr_ref[...], dn,
                                     preferred_element_type=F32) + bl_ref[...]

    grid = (n // bn,)
    return pl.pallas_call(
        body,
        grid=grid,
        in_specs=[
            pl.BlockSpec((bn, h), lambda i: (i, 0)),
            pl.BlockSpec((2, h), lambda i: (0, 0)),
            pl.BlockSpec((1, h), lambda i: (0, 0)),
            pl.BlockSpec((1, h), lambda i: (0, 0)),
            pl.BlockSpec((c, h), lambda i: (0, 0)),
            pl.BlockSpec((c, h), lambda i: (0, 0)),
            pl.BlockSpec((1, c), lambda i: (0, 0)),
        ],
        out_specs=[
            pl.BlockSpec((bn, c), lambda i: (i, 0)),
            pl.BlockSpec((bn, c), lambda i: (i, 0)),
        ],
        out_shape=[
            jax.ShapeDtypeStruct((n, c), F32),
            jax.ShapeDtypeStruct((n, c), F32),
        ],
    )(hpre, stats, gamma, beta, wl2, wr2, bl2)


def _tc_final(agg, deg, z2, bn):
    n, c = z2.shape

    def body(a_ref, d_ref, z_ref, o_ref):
        a = a_ref[0] + a_ref[1]
        o_ref[...] = a / jnp.maximum(d_ref[...], 1.0) + z_ref[...]

    grid = (n // bn,)
    return pl.pallas_call(
        body,
        grid=grid,
        in_specs=[
            pl.BlockSpec((2, bn, c), lambda i: (0, i, 0)),
            pl.BlockSpec((bn, 1), lambda i: (i, 0)),
            pl.BlockSpec((bn, c), lambda i: (i, 0)),
        ],
        out_specs=pl.BlockSpec((bn, c), lambda i: (i, 0)),
        out_shape=jax.ShapeDtypeStruct((n, c), F32),
    )(agg, deg, z2)


def kernel(x, edge_index, Wl1, bl1, Wr1, gamma, beta, Wl2, bl2, Wr2):
    n, d = x.shape
    e = edge_index.shape[1]
    h = Wl1.shape[0]
    c = Wl2.shape[0]

    nch = -(-e // (NW * K))
    nch2 = 2 * nch
    e_pad = NW * K * nch
    rows_per_tile = -(-(n + 1) // (NUM_SUBCORES * 8)) * 8
    n_pad = NUM_SUBCORES * rows_per_tile

    src = edge_index[0]
    dst = edge_index[1]
    pad = e_pad - e
    if pad:
        src = jnp.concatenate([src, jnp.zeros((pad,), jnp.int32)])
        dst = jnp.concatenate([dst, jnp.full((pad,), n, jnp.int32)])
    srcr16 = src.reshape(NUM_SUBCORES, nch2, K)
    dstr16 = dst.reshape(NUM_SUBCORES, nch2, K)
    srcr32 = src.reshape(NW, nch, K)
    dstr32 = dst.reshape(NW, nch, K)

    hw = h // 2
    zeros_hw = jnp.zeros((rows_per_tile, hw), F32)
    zeros_dg = jnp.zeros((rows_per_tile, DEGW), F32)
    zeros_c = jnp.zeros((rows_per_tile, c), F32)
    ones_kd = jnp.ones((K, DEGW), F32)

    bn = 1000 if n % 1000 == 0 else 8 * (n // 8)

    y1a, y1b, z1 = _tc_lin(x, Wl1, Wr1, bl1.reshape(1, h), bn)
    seg1 = _make_sc_layer1(n_pad, rows_per_tile, nch2, hw)
    agg1, degs = seg1(y1a, y1b, srcr16, dstr16, zeros_hw, zeros_dg, ones_kd)

    hpre, stats, deg = _tc_combine_stats(agg1, degs, z1, bn)
    y2, z2 = _tc_bn_lin(hpre, stats, gamma.reshape(1, h), beta.reshape(1, h),
                        Wl2, Wr2, bl2.reshape(1, c), bn)

    seg2 = _make_sc_layer2(n_pad, rows_per_tile, nch, c)
    agg2 = seg2(y2, srcr32, dstr32, zeros_c)
    out = _tc_final(agg2, deg, z2, bn)
    return out
